# initial kernel scaffold (unmeasured)
import jax
import jax.numpy as jnp
from jax import lax
from jax.experimental import pallas as pl
from jax.experimental.pallas import tpu as pltpu

N_DEV = 32
FP8 = jnp.float8_e4m3fn


def kernel(x, w_mat, scale_x, scale_w):
    m_per, k = x.shape
    _, n = w_mat.shape
    n_per = n // N_DEV
    print(f"[kernel] dtypes: x={x.dtype} w={w_mat.dtype} "
          f"sx={scale_x.dtype} shapes x={x.shape} w={w_mat.shape}")

    def body(x_ref, w_ref, sx_ref, sw_ref, out_ref,
             xbuf, wbuf, comm_ref, copy_sems, send_sems, recv_sems):
        me = lax.axis_index("i")
        s = sx_ref[0] * sw_ref[0]

        xbuf[...] = x_ref[...].astype(FP8)

        for d in range(N_DEV):
            j = lax.rem(me + d, N_DEV)
            slot = d % 2
            cp = pltpu.make_async_copy(
                w_ref.at[:, pl.ds(j * n_per, n_per)],
                wbuf.at[slot],
                copy_sems.at[slot],
            )
            cp.start()
            cp.wait()
            chunk = jnp.dot(
                xbuf[...], wbuf[slot].astype(FP8),
                preferred_element_type=jnp.float32,
            ) * s
            if d == 0:
                out_ref[pl.ds(me * m_per, m_per), :] = chunk
            else:
                comm_ref[...] = chunk
                rdma = pltpu.make_async_remote_copy(
                    src_ref=comm_ref,
                    dst_ref=out_ref.at[pl.ds(me * m_per, m_per), :],
                    send_sem=send_sems.at[d],
                    recv_sem=recv_sems.at[d],
                    device_id=(j,),
                    device_id_type=pl.DeviceIdType.MESH,
                )
                rdma.start()
                rdma.wait()

    return pl.pallas_call(
        body,
        out_shape=jax.ShapeDtypeStruct((N_DEV * m_per, n_per), jnp.float32),
        in_specs=[
            pl.BlockSpec(memory_space=pltpu.VMEM),
            pl.BlockSpec(memory_space=pltpu.ANY),
            pl.BlockSpec(memory_space=pltpu.SMEM),
            pl.BlockSpec(memory_space=pltpu.SMEM),
        ],
        out_specs=pl.BlockSpec(memory_space=pltpu.VMEM),
        scratch_shapes=[
            pltpu.VMEM((m_per, k), FP8),
            pltpu.VMEM((2, k, n_per), w_mat.dtype),
            pltpu.VMEM((m_per, n_per), jnp.float32),
            pltpu.SemaphoreType.DMA((2,)),
            pltpu.SemaphoreType.DMA((N_DEV,)),
            pltpu.SemaphoreType.DMA((N_DEV,)),
        ],
        compiler_params=pltpu.CompilerParams(collective_id=0),
    )(x, w_mat, scale_x, scale_w)


# baseline (device time: 241183 ns/iter reference)
import jax
import jax.numpy as jnp
from jax import lax
from jax.experimental import pallas as pl
from jax.experimental.pallas import tpu as pltpu

N_DEV = 32
FP8 = jnp.float8_e4m3fn


def kernel(x, w_mat, scale_x, scale_w):
    m_per, k = x.shape
    _, n = w_mat.shape
    n_per = n // N_DEV
    print(f"[kernel] dtypes: x={x.dtype} w={w_mat.dtype} "
          f"sx={scale_x.dtype} shapes x={x.shape} w={w_mat.shape}")

    def body(x_ref, w_ref, sx_ref, sw_ref, out_ref,
             xbuf, wbuf, comm_ref, copy_sems, send_sems, recv_sems):
        me = lax.axis_index("i")
        s = sx_ref[0] * sw_ref[0]

        xbuf[...] = x_ref[...].astype(FP8)

        for d in range(N_DEV):
            j = lax.rem(me + d, N_DEV)
            slot = d % 2
            cp = pltpu.make_async_copy(
                w_ref.at[:, pl.ds(j * n_per, n_per)],
                wbuf.at[slot],
                copy_sems.at[slot],
            )
            cp.start()
            cp.wait()
            chunk = jnp.dot(
                xbuf[...], wbuf[slot].astype(FP8),
                preferred_element_type=jnp.float32,
            ) * s
            if d == 0:
                out_ref[pl.ds(me * m_per, m_per), :] = chunk
            else:
                comm_ref[...] = chunk
                rdma = pltpu.make_async_remote_copy(
                    src_ref=comm_ref,
                    dst_ref=out_ref.at[pl.ds(me * m_per, m_per), :],
                    send_sem=send_sems.at[d],
                    recv_sem=recv_sems.at[d],
                    device_id=(j,),
                    device_id_type=pl.DeviceIdType.MESH,
                )
                rdma.start()
                rdma.wait()

    return pl.pallas_call(
        body,
        out_shape=jax.ShapeDtypeStruct((N_DEV * m_per, n_per), jnp.float32),
        in_specs=[
            pl.BlockSpec(memory_space=pltpu.VMEM),
            pl.BlockSpec(memory_space=pltpu.MemorySpace.HBM),
            pl.BlockSpec(memory_space=pltpu.SMEM),
            pl.BlockSpec(memory_space=pltpu.SMEM),
        ],
        out_specs=pl.BlockSpec(memory_space=pltpu.VMEM),
        scratch_shapes=[
            pltpu.VMEM((m_per, k), FP8),
            pltpu.VMEM((2, k, n_per), w_mat.dtype),
            pltpu.VMEM((m_per, n_per), jnp.float32),
            pltpu.SemaphoreType.DMA((2,)),
            pltpu.SemaphoreType.DMA((N_DEV,)),
            pltpu.SemaphoreType.DMA((N_DEV,)),
        ],
    )(x, w_mat, scale_x, scale_w)


# device time: 73128 ns/iter; 3.2981x vs baseline; 3.2981x over previous
import jax
import jax.numpy as jnp
from jax import lax
from jax.experimental import pallas as pl
from jax.experimental.pallas import tpu as pltpu

N_DEV = 32
FP8 = jnp.float8_e4m3fn


def kernel(x, w_mat, scale_x, scale_w):
    m_per, k = x.shape
    _, n = w_mat.shape
    n_per = n // N_DEV
    print(f"[kernel] dtypes: x={x.dtype} w={w_mat.dtype} "
          f"sx={scale_x.dtype} shapes x={x.shape} w={w_mat.shape}")

    def body(x_ref, w_ref, sx_ref, sw_ref, out_ref,
             xbuf, wbuf, comm_ref, copy_sems, send_sems, recv_sems):
        me = lax.axis_index("i")
        s = sx_ref[0] * sw_ref[0]

        xbuf[...] = x_ref[...].astype(FP8)

        def w_copy(d, slot):
            j = lax.rem(me + d, N_DEV)
            return pltpu.make_async_copy(
                w_ref.at[:, pl.ds(j * n_per, n_per)],
                wbuf.at[slot],
                copy_sems.at[slot],
            )

        w_copy(0, 0).start()
        rdmas = []
        for d in range(N_DEV):
            slot = d % 2
            if d + 1 < N_DEV:
                w_copy(d + 1, 1 - slot).start()
            w_copy(d, slot).wait()
            chunk = jnp.dot(
                xbuf[...], wbuf[slot].astype(FP8),
                preferred_element_type=jnp.float32,
            ) * s
            if d == 0:
                out_ref[pl.ds(me * m_per, m_per), :] = chunk
            else:
                comm_ref[d] = chunk
                rdma = pltpu.make_async_remote_copy(
                    src_ref=comm_ref.at[d],
                    dst_ref=out_ref.at[pl.ds(me * m_per, m_per), :],
                    send_sem=send_sems.at[d],
                    recv_sem=recv_sems.at[d],
                    device_id=(lax.rem(me + d, N_DEV),),
                    device_id_type=pl.DeviceIdType.MESH,
                )
                rdma.start()
                rdmas.append(rdma)

        for rdma in rdmas:
            rdma.wait()

    return pl.pallas_call(
        body,
        out_shape=jax.ShapeDtypeStruct((N_DEV * m_per, n_per), jnp.float32),
        in_specs=[
            pl.BlockSpec(memory_space=pltpu.VMEM),
            pl.BlockSpec(memory_space=pltpu.MemorySpace.HBM),
            pl.BlockSpec(memory_space=pltpu.SMEM),
            pl.BlockSpec(memory_space=pltpu.SMEM),
        ],
        out_specs=pl.BlockSpec(memory_space=pltpu.VMEM),
        scratch_shapes=[
            pltpu.VMEM((m_per, k), FP8),
            pltpu.VMEM((2, k, n_per), w_mat.dtype),
            pltpu.VMEM((N_DEV, m_per, n_per), jnp.float32),
            pltpu.SemaphoreType.DMA((2,)),
            pltpu.SemaphoreType.DMA((N_DEV,)),
            pltpu.SemaphoreType.DMA((N_DEV,)),
        ],
    )(x, w_mat, scale_x, scale_w)
